# baseline (speedup 1.0000x reference)
import functools

import jax
import jax.numpy as jnp
from jax import lax
from jax.experimental import pallas as pl
from jax.experimental.pallas import tpu as pltpu
from jax.experimental.pallas import tpu_sc as plsc

NC = 2
NS = 16
NW = NC * NS
L = 16


def _sc_body(per_w, x_hbm, u_hbm, m_hbm, out_hbm,
             xv, idx_u, idx_m, rows_u, rows_m, outv,
             sem_x, sem_u, sem_m):
    wid = lax.axis_index("s") * NC + lax.axis_index("c")
    base = wid * per_w

    pltpu.async_copy(x_hbm.at[pl.ds(base, per_w)], xv, sem_x).wait()

    iota = lax.iota(jnp.int32, L)
    zeros = jnp.zeros((L,), jnp.int32)
    ones = jnp.ones((L,), jnp.int32)

    @pl.loop(0, per_w, step=L)
    def _(i):
        rows = iota + i
        idx_u.at[pl.ds(i, L)][...] = plsc.load_gather(xv, [rows, zeros])
        idx_m.at[pl.ds(i, L)][...] = plsc.load_gather(xv, [rows, ones])

    cu = pltpu.async_copy(u_hbm.at[idx_u], rows_u, sem_u)
    cm = pltpu.async_copy(m_hbm.at[idx_m], rows_m, sem_m)
    cu.wait()
    cm.wait()

    @pl.loop(0, per_w, step=L)
    def _(i0):
        acc = jnp.zeros((L,), jnp.float32)
        for j in range(L):
            u = rows_u.at[i0 + j][...]
            m = rows_m.at[i0 + j][...]
            s = jnp.sum(u * m)
            acc = jnp.where(iota == j, s, acc)
        outv.at[pl.ds(i0, L)][...] = acc

    pltpu.sync_copy(outv, out_hbm.at[pl.ds(base, per_w)])


def kernel(x, U, M):
    batch = x.shape[0]
    per_w = batch // NW
    dim = U.shape[1]

    mesh = plsc.VectorSubcoreMesh(core_axis_name="c", subcore_axis_name="s")
    cp = pltpu.CompilerParams(
        needs_layout_passes=False, use_tc_tiling_on_sc=False
    )
    k = pl.kernel(
        functools.partial(_sc_body, per_w),
        out_type=jax.ShapeDtypeStruct((batch,), jnp.float32),
        mesh=mesh,
        scratch_types=[
            pltpu.VMEM((per_w, 2), jnp.int32),
            pltpu.VMEM((per_w,), jnp.int32),
            pltpu.VMEM((per_w,), jnp.int32),
            pltpu.VMEM((per_w, dim), jnp.float32),
            pltpu.VMEM((per_w, dim), jnp.float32),
            pltpu.VMEM((per_w,), jnp.float32),
            pltpu.SemaphoreType.DMA,
            pltpu.SemaphoreType.DMA,
            pltpu.SemaphoreType.DMA,
        ],
        compiler_params=cp,
    )
    out = k(x, U, M)
    return out.reshape(-1, 1)
